# baseline (speedup 1.0000x reference)
import functools

import jax
import jax.numpy as jnp
from jax import lax
from jax.experimental import pallas as pl
from jax.experimental.pallas import tpu as pltpu
from jax.experimental.pallas import tpu_sc as plsc

N = 50000
EMB = 128
H = EMB // 2
HC = H // 2
E = 500000

NTILE = 16
NP = 50048
RPT = NP // NTILE
RB = 136
NBLK = RPT // RB
RTAIL = N - (NP - RB)
G = 2
NIB = 4
C = 304
NCHUNK = 104
EPT = NCHUNK * C
EP = EPT * NTILE
PADE = EP - E

ETYPES = ('uv', 'up', 'vu', 'vt', 'pu', 'pt', 'tv', 'tp')
REV = {'uv': 'vu', 'up': 'pu', 'vu': 'uv', 'vt': 'tv',
       'pu': 'up', 'pt': 'tp', 'tv': 'vt', 'tp': 'pt'}
COL_OFF = {'uv': 0, 'up': H, 'vu': 0, 'vt': H,
           'pu': 0, 'pt': H, 'tv': 0, 'tp': H}
NTYPES = ('u', 'v', 'p', 't')
HS = {'vu': 0, 'pu': 1, 'uv': 0, 'tv': 1, 'up': 0, 'tp': 1,
      'vt': 0, 'pt': 1}

_mesh = plsc.VectorSubcoreMesh(core_axis_name="c", subcore_axis_name="s")

_out_types = ([jax.ShapeDtypeStruct((2, NP, HC), jnp.float32)] * 8 +
              [jax.ShapeDtypeStruct((N, EMB), jnp.float32)] * 4 +
              [jax.ShapeDtypeStruct((8, NP), jnp.float32)])

_scratch = (
    [pltpu.VMEM_SHARED((NP, HC), jnp.float32),
     pltpu.VMEM_SHARED((NP,), jnp.float32)] +
    [pltpu.VMEM((2, C), jnp.int32)] * NIB +
    [pltpu.VMEM((C,), jnp.int32)] * G +
    [pltpu.VMEM((C, HC), jnp.float32)] * G +
    [pltpu.VMEM((C,), jnp.float32),
     pltpu.VMEM((RB, HC), jnp.float32),
     pltpu.VMEM((RB + 16,), jnp.float32),
     pltpu.VMEM((RB,), jnp.float32),
     pltpu.VMEM((RB,), jnp.int32)] +
    [pltpu.SemaphoreType.DMA] * G +
    [pltpu.SemaphoreType.DMA] * G +
    [pltpu.SemaphoreType.DMA] * NIB
)


@functools.partial(pl.kernel, mesh=_mesh, out_type=_out_types,
                   scratch_types=_scratch,
                   compiler_params=pltpu.CompilerParams(
                       use_tc_tiling_on_sc=False))
def _sc_aggr(*refs):
    embf = refs[0:4]
    edges = refs[4:12]
    ones_h, zrows_h, zrpt_h = refs[12:15]
    out1 = refs[15:23]
    fin = refs[23:27]
    recip_h = refs[27]
    acc, cnt = refs[28:30]
    base = 30
    idxb = refs[base:base + NIB]; base += NIB
    sadj = refs[base:base + G]; base += G
    rows = refs[base:base + G]; base += G
    (ones_v, zb, rcb, z1b, bidx) = refs[base:base + 5]; base += 5
    gsem = refs[base:base + G]; base += G
    ssem = refs[base:base + G]; base += G
    isem = refs[base:base + NIB]
    accb, bb = rows[0], rows[1]

    c = lax.axis_index("c")
    s = lax.axis_index("s")
    row0 = s * RPT
    ebase = s * EPT

    pltpu.sync_copy(zrows_h, zb)
    pltpu.sync_copy(zrpt_h, z1b)
    pltpu.sync_copy(ones_h, ones_v)

    def zero_blk(b, _):
        r0 = row0 + b * RB
        pltpu.sync_copy(zb, acc.at[pl.ds(r0, RB), :])
        pltpu.sync_copy(z1b, cnt.at[pl.ds(r0, RB)])
        return 0
    lax.fori_loop(0, NBLK, zero_blk, 0)
    plsc.subcore_barrier()

    for layer in (1, 2):
        for ei, e in enumerate(ETYPES):
            rev_i = ETYPES.index(REV[e])
            edg = edges[ei]
            tgt = NTYPES.index(e[1])
            col0 = HS[e] * H + c * HC
            if layer == 1:
                tab = embf[NTYPES.index(e[0])]
                qsrc = jnp.broadcast_to(COL_OFF[e] // 32 + c, (16,))
            else:
                tab = out1[rev_i]

            def idx_fire(i, t):
                off = ebase + i * C
                pltpu.async_copy(edg.at[:, pl.ds(off, C)], idxb[t], isem[t])

            def idx_wait(i, t):
                off = ebase + i * C
                pltpu.make_async_copy(edg.at[:, pl.ds(off, C)], idxb[t],
                                      isem[t]).wait()

            def launch(i, s, t, tab=tab, layer=layer,
                       qsrc=(qsrc if layer == 1 else None)):
                idx_wait(i, t)
                if layer == 1:
                    def adj(j, _):
                        v = idxb[t][0, pl.ds(j * 16, 16)]
                        sadj[s][pl.ds(j * 16, 16)] = v * 4 + qsrc
                        return 0
                    lax.fori_loop(0, C // 16, adj, 0)
                    pltpu.async_copy(tab.at[sadj[s]], rows[s], gsem[s])
                else:
                    pltpu.async_copy(tab.at[c].at[idxb[t].at[0]],
                                     rows[s], gsem[s])

            def wait_g(s, t, tab=tab, layer=layer):
                if layer == 1:
                    pltpu.make_async_copy(tab.at[sadj[s]], rows[s],
                                          gsem[s]).wait()
                else:
                    pltpu.make_async_copy(tab.at[c].at[idxb[t].at[0]],
                                          rows[s], gsem[s]).wait()

            def fire_scatter(s, t, layer=layer):
                pltpu.async_copy(rows[s], acc.at[idxb[t].at[1]], ssem[s],
                                 add=True)
                if layer == 1:
                    pltpu.async_copy(ones_v, cnt.at[idxb[t].at[1]], ssem[s],
                                     add=True)

            def drain_scatter(s, t, layer=layer):
                pltpu.make_async_copy(rows[s], acc.at[idxb[t].at[1]],
                                      ssem[s]).wait()
                if layer == 1:
                    pltpu.make_async_copy(ones_v, cnt.at[idxb[t].at[1]],
                                          ssem[s]).wait()

            def step(i, s, t, drain=True, finish_prev=True, guard=False):
                if drain:
                    drain_scatter(s, (t + 2) % NIB)
                launch(i, s, t)
                if guard:
                    @pl.when(i + 2 < NCHUNK)
                    def _():
                        idx_fire(i + 2, (t + 2) % NIB)
                else:
                    idx_fire(i + 2, (t + 2) % NIB)
                if finish_prev:
                    wait_g(s ^ 1, (t + 3) % NIB)
                    fire_scatter(s ^ 1, (t + 3) % NIB)

            idx_fire(0, 0)
            idx_fire(1, 1)
            step(0, 0, 0, drain=False, finish_prev=False)
            step(1, 1, 1, drain=False)
            step(2, 0, 2)
            step(3, 1, 3)

            def gs_group(k, _):
                for j in range(4):
                    step(4 * k + j, j % 2, j, guard=True)
                return 0
            lax.fori_loop(1, NCHUNK // 4, gs_group, 0)
            wait_g(1, 3)
            fire_scatter(1, 3)
            drain_scatter(0, 2)
            drain_scatter(1, 3)
            plsc.subcore_barrier()

            if layer == 1:
                basef = embf[tgt]
                qb = jnp.broadcast_to(COL_OFF[REV[e]] // 32 + c, (16,))

            def scale_blk(b, _, ei=ei, tgt=tgt, col0=col0, layer=layer):
                r0 = row0 + b * RB
                if layer == 1:
                    pltpu.sync_copy(cnt.at[pl.ds(r0, RB)],
                                    rcb.at[pl.ds(0, RB)])
                else:
                    pltpu.sync_copy(recip_h.at[ei].at[pl.ds(r0, RB)],
                                    rcb.at[pl.ds(0, RB)])
                pltpu.sync_copy(acc.at[pl.ds(r0, RB), :],
                                accb.at[pl.ds(0, RB), :])
                if layer == 1:
                    def bix(j, _):
                        off = jnp.minimum(j * 16, RB - 16)
                        sl = lax.broadcasted_iota(jnp.int32, (16,), 0)
                        bidx[pl.ds(off, 16)] = (r0 + off) * 4 + sl * 4 + qb
                        return 0
                    lax.fori_loop(0, (RB + 15) // 16, bix, 0)
                    pltpu.async_copy(basef.at[bidx],
                                     bb.at[pl.ds(0, RB), :], gsem[0]).wait()
                else:
                    @pl.when(r0 + RB <= N)
                    def _():
                        pltpu.sync_copy(
                            fin[tgt].at[pl.ds(r0, RB), pl.ds(col0, HC)],
                            bb.at[pl.ds(0, RB), :])

                    @pl.when(r0 + RB > N)
                    def _():
                        pltpu.sync_copy(
                            fin[tgt].at[pl.ds(r0, RTAIL), pl.ds(col0, HC)],
                            bb.at[pl.ds(0, RTAIL), :])

                if layer == 1:
                    def rcp_body(j, _):
                        v = rcb[pl.ds(j * 16, 16)]
                        rcb[pl.ds(j * 16, 16)] = 1.0 / jnp.maximum(v, 1.0)
                        return 0
                    lax.fori_loop(0, (RB + 15) // 16, rcp_body, 0)
                    pltpu.sync_copy(rcb.at[pl.ds(0, RB)],
                                    recip_h.at[ei].at[pl.ds(r0, RB)])

                def row_body(r, _, layer=layer):
                    rcv = rcb[pl.ds(r, 16)][0]
                    if layer == 1:
                        bc = jnp.broadcast_to(rcv, (16,))
                        for h in (0, 16):
                            m = accb[r, pl.ds(h, 16)] * bc
                            accb[r, pl.ds(h, 16)] = m
                            bb[r, pl.ds(h, 16)] = bb[r, pl.ds(h, 16)] + m * 0.5
                    else:
                        bc3 = jnp.broadcast_to(rcv * (1.0 / 3.0), (16,))
                        for h in (0, 16):
                            a = accb[r, pl.ds(h, 16)]
                            bb[r, pl.ds(h, 16)] = bb[r, pl.ds(h, 16)] + a * bc3
                    return 0
                lax.fori_loop(0, RB, row_body, 0)

                if layer == 1:
                    pltpu.sync_copy(accb.at[pl.ds(0, RB), :],
                                    out1[ei].at[c].at[pl.ds(r0, RB), :])

                @pl.when(r0 + RB <= N)
                def _():
                    pltpu.sync_copy(
                        bb.at[pl.ds(0, RB), :],
                        fin[tgt].at[pl.ds(r0, RB), pl.ds(col0, HC)])

                @pl.when(r0 + RB > N)
                def _():
                    pltpu.sync_copy(
                        bb.at[pl.ds(0, RTAIL), :],
                        fin[tgt].at[pl.ds(r0, RTAIL), pl.ds(col0, HC)])

                pltpu.sync_copy(zb, acc.at[pl.ds(r0, RB), :])
                if layer == 1:
                    pltpu.sync_copy(z1b, cnt.at[pl.ds(r0, RB)])
                return 0
            lax.fori_loop(0, NBLK, scale_blk, 0)
            plsc.subcore_barrier()


def kernel(user_emb, video_emb, publisher_emb, tag_emb,
           edge_index_uv, edge_index_up, edge_index_vu, edge_index_vt,
           edge_index_pu, edge_index_pt, edge_index_tv, edge_index_tp):
    embs = {'u': user_emb, 'v': video_emb, 'p': publisher_emb, 't': tag_emb}
    ei = {'uv': edge_index_uv, 'up': edge_index_up, 'vu': edge_index_vu,
          'vt': edge_index_vt, 'pu': edge_index_pu, 'pt': edge_index_pt,
          'tv': edge_index_tv, 'tp': edge_index_tp}

    embf = [jnp.pad(embs[t], ((0, NP - N), (0, 0))).reshape(NP * 4, HC)
            for t in NTYPES]
    edges = [jnp.pad(ei[e], ((0, 0), (0, PADE)), constant_values=N)
             for e in ETYPES]

    ones_c = jnp.ones((C,), jnp.float32)
    zrows = jnp.zeros((RB, HC), jnp.float32)
    zrpt = jnp.zeros((RB,), jnp.float32)

    outs = _sc_aggr(*embf, *edges, ones_c, zrows, zrpt)
    return (outs[8], outs[9], outs[10], outs[11])
